# baseline (speedup 1.0000x reference)
import functools

import jax
import jax.numpy as jnp
import numpy as np
from jax import lax
from jax.experimental import pallas as pl
from jax.experimental.pallas import tpu as pltpu
from jax.experimental.pallas import tpu_sc as plsc

_BS, _K, _N = 16, 8, 128
_H, _W = 256, 256
_OH, _OW = 512, 256
_C = _BS * _K
_CHUNK_ROWS = 128
_NCHUNK = _OH // _CHUNK_ROWS
_L = 16


def _rev(x):
    return lax.rev(x, (0,))


def _cswap(ak, av, bk, bv):
    cond = ak > bk
    lk = jnp.where(cond, bk, ak)
    lv = jnp.where(cond, bv, av)
    hk = jnp.where(cond, ak, bk)
    hv = jnp.where(cond, av, bv)
    return lk, lv, hk, hv


def _bitonic_merge(seq):
    m = len(seq)
    if m == 1:
        k, v = seq[0]
        sk, sv = plsc.sort_key_val(k, v)
        return [(sk, sv)]
    half = m // 2
    lo, hi = [], []
    for i in range(half):
        ak, av = seq[i]
        bk, bv = seq[i + half]
        lk, lv, hk, hv = _cswap(ak, av, bk, bv)
        lo.append((lk, lv))
        hi.append((hk, hv))
    return _bitonic_merge(lo) + _bitonic_merge(hi)


def _merge_sorted(a, b):
    br = [(_rev(k), _rev(v)) for (k, v) in reversed(b)]
    lo, hi = [], []
    for (ak, av), (bk, bv) in zip(a, br):
        lk, lv, hk, hv = _cswap(ak, av, bk, bv)
        lo.append((lk, lv))
        hi.append((hk, hv))
    return _bitonic_merge(lo) + _bitonic_merge(hi)


def _paint_call(ts, v, const01):
    mesh = plsc.VectorSubcoreMesh(core_axis_name="c", subcore_axis_name="s")
    info = plsc.get_sparse_core_info()
    nc = info.num_cores
    nw = nc * info.num_subcores
    curves_per_w = _C // nw
    eps = float(np.spacing(np.finfo(np.float32).eps))

    @functools.partial(
        pl.kernel,
        out_type=jax.ShapeDtypeStruct((_BS, _K, _OH, _OW), jnp.float32),
        mesh=mesh,
        compiler_params=pltpu.CompilerParams(needs_layout_passes=False),
        scratch_types=[
            pltpu.VMEM((3, _CHUNK_ROWS, _OW), jnp.float32),
            pltpu.VMEM((_N,), jnp.float32),
            pltpu.VMEM((_N,), jnp.float32),
            pltpu.VMEM((_N,), jnp.float32),
            pltpu.VMEM((_N,), jnp.float32),
            pltpu.VMEM((2 * _H,), jnp.int32),
            pltpu.SemaphoreType.DMA,
            pltpu.SemaphoreType.DMA,
            pltpu.SemaphoreType.DMA,
        ],
    )
    def body(ts_hbm, v_hbm, const_hbm, out_hbm, buf, tbuf, vbuf, tso, vso,
             viv, sem0, sem1, sem2):
        wid = lax.axis_index("s") * nc + lax.axis_index("c")
        pltpu.sync_copy(const_hbm, buf.at[0])
        pltpu.sync_copy(const_hbm, buf.at[1])
        pltpu.sync_copy(const_hbm, buf.at[2])

        lanes = lax.iota(jnp.int32, _L)

        def scatter_pass(slot, vbase, r0, sgn):
            slot_v = lanes - lanes + slot

            def body_j(j, carry):
                rloc = j * _L + lanes
                r = r0 + rloc
                m = lax.shift_right_logical(r, 1)
                is_odd = lax.bitwise_and(r, 1) == 1
                ya = jnp.where(is_odd, m, jnp.maximum(m - 1, 0))
                yb = jnp.where(is_odd, jnp.minimum(m + 1, _H - 1), m)
                wa = jnp.where(is_odd, jnp.float32(0.675 * sgn),
                               jnp.float32(0.225 * sgn))
                wb = jnp.where(is_odd, jnp.float32(0.225 * sgn),
                               jnp.float32(0.675 * sgn))
                ca = plsc.load_gather(viv, [vbase + ya])
                cb = plsc.load_gather(viv, [vbase + yb])
                plsc.addupdate_scatter(buf, [slot_v, rloc, ca], wa)
                plsc.addupdate_scatter(buf, [slot_v, rloc, cb], wb)
                return carry
            lax.fori_loop(0, _CHUNK_ROWS // _L, body_j, 0)

        def drain(slot):
            @pl.when(slot == 0)
            def _():
                pltpu.make_async_copy(const_hbm, buf.at[0], sem0).wait()

            @pl.when(slot == 1)
            def _():
                pltpu.make_async_copy(const_hbm, buf.at[1], sem1).wait()

            @pl.when(slot == 2)
            def _():
                pltpu.make_async_copy(const_hbm, buf.at[2], sem2).wait()

        def per_step(g, carry):
            k = lax.div(g, _NCHUNK)
            ch = lax.rem(g, _NCHUNK)
            c = wid * curves_per_w + k
            b = lax.div(c, _K)
            kk = lax.rem(c, _K)
            slot = lax.rem(g, 3)
            vbase = lax.rem(k, 2) * _H
            r0 = ch * _CHUNK_ROWS

            @pl.when(ch == 0)
            def _sort_and_search():
                pltpu.sync_copy(ts_hbm.at[c], tbuf)
                pltpu.sync_copy(v_hbm.at[c], vbuf)
                runs = []
                for i in range(_N // _L):
                    ki = tbuf[pl.ds(i * _L, _L)]
                    vi_ = vbuf[pl.ds(i * _L, _L)]
                    runs.append([plsc.sort_key_val(ki, vi_)])
                while len(runs) > 1:
                    runs = [_merge_sorted(runs[i], runs[i + 1])
                            for i in range(0, len(runs), 2)]
                for i, (ki, vi_) in enumerate(runs[0]):
                    tso[pl.ds(i * _L, _L)] = ki
                    vso[pl.ds(i * _L, _L)] = vi_
                zero16 = lanes - lanes
                t_first = runs[0][0][0][0]
                v_first = runs[0][0][1][0]
                t_last = runs[0][-1][0][_L - 1]
                v_last = runs[0][-1][1][_L - 1]
                for qi in range(_H // _L):
                    tqi = lanes + qi * _L
                    tqf = tqi.astype(jnp.float32)
                    cnt = zero16
                    for s in (64, 32, 16, 8, 4, 2, 1):
                        val = plsc.load_gather(tso, [cnt + (s - 1)])
                        cnt = cnt + jnp.where(val <= tqf, s, 0)
                    i1 = jnp.clip(cnt, 1, _N - 1)
                    i0 = i1 - 1
                    tlo = plsc.load_gather(tso, [i0])
                    vlo = plsc.load_gather(vso, [i0])
                    thi = plsc.load_gather(tso, [i1])
                    vhi = plsc.load_gather(vso, [i1])
                    dx = thi - tlo
                    dx0 = jnp.abs(dx) <= eps
                    f = jnp.where(
                        dx0, vlo,
                        vlo + ((tqf - tlo)
                               / jnp.where(dx0, jnp.float32(1.0), dx))
                        * (vhi - vlo))
                    f = jnp.where(tqf < t_first, v_first, f)
                    f = jnp.where(tqf > t_last, v_last, f)
                    fc = jnp.minimum(jnp.maximum(f + 0.5, 0.0), float(_W - 1))
                    plsc.store_scatter(viv, [vbase + qi * _L + lanes],
                                       fc.astype(jnp.int32))

            @pl.when(g >= 3)
            def _retire():
                drain(slot)
                g2 = g - 3
                k2 = lax.div(g2, _NCHUNK)
                ch2 = lax.rem(g2, _NCHUNK)
                scatter_pass(slot, lax.rem(k2, 2) * _H,
                             ch2 * _CHUNK_ROWS, -1.0)

            scatter_pass(slot, vbase, r0, 1.0)
            dst = out_hbm.at[b, kk, pl.ds(r0, _CHUNK_ROWS)]

            @pl.when(slot == 0)
            def _():
                pltpu.async_copy(buf.at[0], dst, sem0)

            @pl.when(slot == 1)
            def _():
                pltpu.async_copy(buf.at[1], dst, sem1)

            @pl.when(slot == 2)
            def _():
                pltpu.async_copy(buf.at[2], dst, sem2)

            return carry

        lax.fori_loop(0, curves_per_w * _NCHUNK, per_step, 0)
        pltpu.make_async_copy(const_hbm, buf.at[0], sem0).wait()
        pltpu.make_async_copy(const_hbm, buf.at[1], sem1).wait()
        pltpu.make_async_copy(const_hbm, buf.at[2], sem2).wait()

    return body(ts, v, const01)


def kernel(VelPoints, VMM):
    t = VelPoints[..., 0].reshape(_C, _N) / np.float32(1.0 / (_H - 1))
    ts = jnp.where(t > 0, t, jnp.float32(1e9))
    vmin = jnp.repeat(VMM[:, 0], _K).reshape(_C, 1)
    vmax = jnp.repeat(VMM[:, 1], _K).reshape(_C, 1)
    stepv = (vmax - vmin) / np.float32(_W - 1)
    v = (VelPoints[..., 1].reshape(_C, _N) - vmin) / stepv
    const01 = jnp.full((_CHUNK_ROWS, _OW), 0.01, jnp.float32)
    return _paint_call(ts, v, const01)


# speedup vs baseline: 1.1019x; 1.1019x over previous
import functools

import jax
import jax.numpy as jnp
import numpy as np
from jax import lax
from jax.experimental import pallas as pl
from jax.experimental.pallas import tpu as pltpu
from jax.experimental.pallas import tpu_sc as plsc

_BS, _K, _N = 16, 8, 128
_H, _W = 256, 256
_OH, _OW = 512, 256
_C = _BS * _K
_CHUNK_ROWS = 128
_NCHUNK = _OH // _CHUNK_ROWS
_L = 16


def _rev(x):
    return lax.rev(x, (0,))


def _cswap(ak, av, bk, bv):
    cond = ak > bk
    lk = jnp.where(cond, bk, ak)
    lv = jnp.where(cond, bv, av)
    hk = jnp.where(cond, ak, bk)
    hv = jnp.where(cond, av, bv)
    return lk, lv, hk, hv


def _bitonic_merge(seq):
    m = len(seq)
    if m == 1:
        k, v = seq[0]
        sk, sv = plsc.sort_key_val(k, v)
        return [(sk, sv)]
    half = m // 2
    lo, hi = [], []
    for i in range(half):
        ak, av = seq[i]
        bk, bv = seq[i + half]
        lk, lv, hk, hv = _cswap(ak, av, bk, bv)
        lo.append((lk, lv))
        hi.append((hk, hv))
    return _bitonic_merge(lo) + _bitonic_merge(hi)


def _merge_sorted(a, b):
    br = [(_rev(k), _rev(v)) for (k, v) in reversed(b)]
    lo, hi = [], []
    for (ak, av), (bk, bv) in zip(a, br):
        lk, lv, hk, hv = _cswap(ak, av, bk, bv)
        lo.append((lk, lv))
        hi.append((hk, hv))
    return _bitonic_merge(lo) + _bitonic_merge(hi)


def _paint_call(ts, v, const01):
    mesh = plsc.VectorSubcoreMesh(core_axis_name="c", subcore_axis_name="s")
    info = plsc.get_sparse_core_info()
    nc = info.num_cores
    nw = nc * info.num_subcores
    curves_per_w = _C // nw
    eps = float(np.spacing(np.finfo(np.float32).eps))

    @functools.partial(
        pl.kernel,
        out_type=jax.ShapeDtypeStruct((_BS, _K, _OH, _OW), jnp.float32),
        mesh=mesh,
        compiler_params=pltpu.CompilerParams(needs_layout_passes=False),
        scratch_types=[
            pltpu.VMEM((2, _CHUNK_ROWS, _OW), jnp.float32),
            pltpu.VMEM((_N,), jnp.float32),
            pltpu.VMEM((_N,), jnp.float32),
            pltpu.VMEM((_N,), jnp.float32),
            pltpu.VMEM((_N,), jnp.float32),
            pltpu.VMEM((2 * _H,), jnp.int32),
            pltpu.SemaphoreType.DMA,
            pltpu.SemaphoreType.DMA,
        ],
    )
    def body(ts_hbm, v_hbm, const_hbm, out_hbm, buf, tbuf, vbuf, tso, vso,
             viv, sem0, sem1):
        wid = lax.axis_index("s") * nc + lax.axis_index("c")
        pltpu.sync_copy(const_hbm, buf.at[0])
        pltpu.sync_copy(const_hbm, buf.at[1])

        lanes = lax.iota(jnp.int32, _L)

        def scatter_pass(slot, vbase, r0, sgn):
            slot_v = lanes - lanes + slot

            def body_j(j, carry):
                rloc = j * _L + lanes
                r = r0 + rloc
                m = lax.shift_right_logical(r, 1)
                is_odd = lax.bitwise_and(r, 1) == 1
                ya = jnp.where(is_odd, m, jnp.maximum(m - 1, 0))
                yb = jnp.where(is_odd, jnp.minimum(m + 1, _H - 1), m)
                wa = jnp.where(is_odd, jnp.float32(0.675 * sgn),
                               jnp.float32(0.225 * sgn))
                wb = jnp.where(is_odd, jnp.float32(0.225 * sgn),
                               jnp.float32(0.675 * sgn))
                ca = plsc.load_gather(viv, [vbase + ya])
                cb = plsc.load_gather(viv, [vbase + yb])
                plsc.addupdate_scatter(buf, [slot_v, rloc, ca], wa)
                plsc.addupdate_scatter(buf, [slot_v, rloc, cb], wb)
                return carry
            lax.fori_loop(0, _CHUNK_ROWS // _L, body_j, 0)

        def drain(slot):
            @pl.when(slot == 0)
            def _():
                pltpu.make_async_copy(const_hbm, buf.at[0], sem0).wait()

            @pl.when(slot == 1)
            def _():
                pltpu.make_async_copy(const_hbm, buf.at[1], sem1).wait()

        def per_step(g, carry):
            k = lax.div(g, _NCHUNK)
            ch = lax.rem(g, _NCHUNK)
            c = wid * curves_per_w + k
            b = lax.div(c, _K)
            kk = lax.rem(c, _K)
            slot = lax.rem(g, 2)
            vbase = lax.rem(k, 2) * _H
            r0 = ch * _CHUNK_ROWS

            @pl.when(ch == 0)
            def _sort_and_search():
                pltpu.sync_copy(ts_hbm.at[c], tbuf)
                pltpu.sync_copy(v_hbm.at[c], vbuf)
                runs = []
                for i in range(_N // _L):
                    ki = tbuf[pl.ds(i * _L, _L)]
                    vi_ = vbuf[pl.ds(i * _L, _L)]
                    runs.append([plsc.sort_key_val(ki, vi_)])
                while len(runs) > 1:
                    runs = [_merge_sorted(runs[i], runs[i + 1])
                            for i in range(0, len(runs), 2)]
                for i, (ki, vi_) in enumerate(runs[0]):
                    tso[pl.ds(i * _L, _L)] = ki
                    vso[pl.ds(i * _L, _L)] = vi_
                zero16 = lanes - lanes
                t_first = runs[0][0][0][0]
                v_first = runs[0][0][1][0]
                t_last = runs[0][-1][0][_L - 1]
                v_last = runs[0][-1][1][_L - 1]
                for qi in range(_H // _L):
                    tqi = lanes + qi * _L
                    tqf = tqi.astype(jnp.float32)
                    cnt = zero16
                    for s in (64, 32, 16, 8, 4, 2, 1):
                        val = plsc.load_gather(tso, [cnt + (s - 1)])
                        cnt = cnt + jnp.where(val <= tqf, s, 0)
                    i1 = jnp.clip(cnt, 1, _N - 1)
                    i0 = i1 - 1
                    tlo = plsc.load_gather(tso, [i0])
                    vlo = plsc.load_gather(vso, [i0])
                    thi = plsc.load_gather(tso, [i1])
                    vhi = plsc.load_gather(vso, [i1])
                    dx = thi - tlo
                    dx0 = jnp.abs(dx) <= eps
                    f = jnp.where(
                        dx0, vlo,
                        vlo + ((tqf - tlo)
                               / jnp.where(dx0, jnp.float32(1.0), dx))
                        * (vhi - vlo))
                    f = jnp.where(tqf < t_first, v_first, f)
                    f = jnp.where(tqf > t_last, v_last, f)
                    fc = jnp.minimum(jnp.maximum(f + 0.5, 0.0), float(_W - 1))
                    plsc.store_scatter(viv, [vbase + qi * _L + lanes],
                                       fc.astype(jnp.int32))

            @pl.when(g >= 2)
            def _retire():
                drain(slot)
                g2 = g - 2
                k2 = lax.div(g2, _NCHUNK)
                ch2 = lax.rem(g2, _NCHUNK)
                scatter_pass(slot, lax.rem(k2, 2) * _H,
                             ch2 * _CHUNK_ROWS, -1.0)

            scatter_pass(slot, vbase, r0, 1.0)
            dst = out_hbm.at[b, kk, pl.ds(r0, _CHUNK_ROWS)]

            @pl.when(slot == 0)
            def _():
                pltpu.async_copy(buf.at[0], dst, sem0)

            @pl.when(slot == 1)
            def _():
                pltpu.async_copy(buf.at[1], dst, sem1)

            return carry

        lax.fori_loop(0, curves_per_w * _NCHUNK, per_step, 0)
        pltpu.make_async_copy(const_hbm, buf.at[0], sem0).wait()
        pltpu.make_async_copy(const_hbm, buf.at[1], sem1).wait()

    return body(ts, v, const01)


def kernel(VelPoints, VMM):
    t = VelPoints[..., 0].reshape(_C, _N) / np.float32(1.0 / (_H - 1))
    ts = jnp.where(t > 0, t, jnp.float32(1e9))
    vmin = jnp.repeat(VMM[:, 0], _K).reshape(_C, 1)
    vmax = jnp.repeat(VMM[:, 1], _K).reshape(_C, 1)
    stepv = (vmax - vmin) / np.float32(_W - 1)
    v = (VelPoints[..., 1].reshape(_C, _N) - vmin) / stepv
    const01 = jnp.full((_CHUNK_ROWS, _OW), 0.01, jnp.float32)
    return _paint_call(ts, v, const01)
